# baseline (speedup 1.0000x reference)
import functools

import jax
import jax.numpy as jnp
from jax import lax
from jax.experimental import pallas as pl
from jax.experimental.pallas import tpu as pltpu
from jax.experimental.pallas import tpu_sc as plsc

_NC = 2
_NS = 16
_NW = _NC * _NS


@functools.lru_cache(maxsize=None)
def _make_sc_gather(B: int, D: int):
    b_per_w = B // _NW
    mesh = plsc.VectorSubcoreMesh(core_axis_name="c", subcore_axis_name="s")

    @functools.partial(
        pl.kernel,
        mesh=mesh,
        compiler_params=pltpu.CompilerParams(use_tc_tiling_on_sc=False),
        out_type=jax.ShapeDtypeStruct((B, D), jnp.float32),
        scratch_types=[
            pltpu.VMEM((b_per_w,), jnp.int32),
            pltpu.VMEM((b_per_w, D), jnp.float32),
            pltpu.SemaphoreType.DMA,
        ],
    )
    def gather_k(idx_hbm, table_hbm, out_hbm, idx_v, rows_v, sem):
        wid = lax.axis_index("s") * _NC + lax.axis_index("c")
        base = wid * b_per_w
        pltpu.sync_copy(idx_hbm.at[pl.ds(base, b_per_w)], idx_v)
        pltpu.async_copy(table_hbm.at[idx_v], rows_v, sem).wait()
        pltpu.sync_copy(rows_v, out_hbm.at[pl.ds(base, b_per_w)])

    return gather_k


def _tc_body(ea_ref, eb_ref, dose_ref, w1a_ref, w1b_ref, w1d_ref, b1_ref,
             w2_ref, b2_ref, w3t_ref, b3_ref, oa_ref, ob_ref):
    ea = ea_ref[...]
    eb = eb_ref[...]
    h = jnp.dot(ea, w1a_ref[...], preferred_element_type=jnp.float32)
    h += jnp.dot(eb, w1b_ref[...], preferred_element_type=jnp.float32)
    h = jnp.maximum(h + dose_ref[...] * w1d_ref[...] + b1_ref[...], 0.0)
    h = jnp.dot(h, w2_ref[...], preferred_element_type=jnp.float32)
    h = jnp.maximum(h + b2_ref[...], 0.0)
    s = jnp.sum(h * w3t_ref[...], axis=1, keepdims=True) + b3_ref[0, 0]
    oa_ref[...] = ea * s
    ob_ref[...] = eb * s


@functools.lru_cache(maxsize=None)
def _make_tc_mlp(B: int, H: int, W: int, BLK: int):
    grid = (B // BLK,)
    full = lambda i: (0, 0)
    blk = lambda i: (i, 0)
    return pl.pallas_call(
        _tc_body,
        grid=grid,
        in_specs=[
            pl.BlockSpec((BLK, H), blk),
            pl.BlockSpec((BLK, H), blk),
            pl.BlockSpec((BLK, 1), blk),
            pl.BlockSpec((H, W), full),
            pl.BlockSpec((H, W), full),
            pl.BlockSpec((1, W), full),
            pl.BlockSpec((1, W), full),
            pl.BlockSpec((W, W), full),
            pl.BlockSpec((1, W), full),
            pl.BlockSpec((1, W), full),
            pl.BlockSpec((1, 1), full),
        ],
        out_specs=[pl.BlockSpec((BLK, H), blk), pl.BlockSpec((BLK, H), blk)],
        out_shape=[jax.ShapeDtypeStruct((B, H), jnp.float32),
                   jax.ShapeDtypeStruct((B, H), jnp.float32)],
    )


def kernel(batch_idx, dose, table, W1, b1, W2, b2, W3, b3):
    B = batch_idx.shape[0]
    V, D = table.shape
    W = W2.shape[0]
    H = D // 2
    idx = batch_idx.astype(jnp.int32)
    gather = _make_sc_gather(B, H)
    ea = gather(idx, table[:, :H])
    eb = gather(idx, table[:, H:])
    oa, ob = _make_tc_mlp(B, H, W, 2048)(
        ea,
        eb,
        dose,
        W1[:H],
        W1[H:D],
        W1[D:D + 1],
        b1.reshape(1, W),
        W2,
        b2.reshape(1, W),
        W3.reshape(1, W),
        b3.reshape(1, 1),
    )
    return jnp.concatenate([oa, ob], axis=1)


# speedup vs baseline: 5.0761x; 5.0761x over previous
import functools

import jax
import jax.numpy as jnp
from jax import lax
from jax.experimental import pallas as pl
from jax.experimental.pallas import tpu as pltpu
from jax.experimental.pallas import tpu_sc as plsc

_NC = 2
_NS = 16
_NW = _NC * _NS
_WIN = 2048
_RING = 4
_RSLOTS = 32


def _walk_run(col, buf, win_idx_v, win_ord_v, idxs_hbm, ord_hbm, out_hbm,
              rbuf, out_sem, state):
    def cond(st):
        return st[3] == 1

    def body(st):
        ptr, wb, nfired, _ = st
        need = ptr + 16 > wb + _WIN

        def slide(_):
            wb2 = pl.multiple_of((ptr // 8) * 8, 8)
            pltpu.sync_copy(idxs_hbm.at[pl.ds(wb2, _WIN)], win_idx_v)
            pltpu.sync_copy(ord_hbm.at[pl.ds(wb2, _WIN)], win_ord_v)
            return wb2

        wb = lax.cond(need, slide, lambda _: wb, 0)
        off = ptr - wb
        iv = win_idx_v[pl.ds(off, 16)]
        pv = win_ord_v[pl.ds(off, 16)]
        q = iv >> 7
        mv = iv & 127
        mski = (q == col).astype(jnp.int32)
        rank = jnp.int32(0)
        for l in range(16):
            take = mski[l] == 1
            slot = lax.rem(nfired + rank, jnp.int32(_RSLOTS))

            @pl.when(take)
            def _():
                @pl.when(nfired + rank >= _RSLOTS)
                def _():
                    pltpu.make_async_copy(out_hbm.at[pl.ds(0, 64)],
                                          rbuf.at[pl.ds(0, 64)],
                                          out_sem).wait()
                m16 = jnp.full((16,), mv[l], jnp.int32)
                base = slot * 64
                for k in range(4):
                    fi = lax.iota(jnp.int32, 16) + 16 * k
                    vals = plsc.load_gather(buf, [fi, m16])
                    rbuf[pl.ds(base + 16 * k, 16)] = vals
                dst = pl.multiple_of(pv[l] * 64, 8)
                pltpu.async_copy(rbuf.at[pl.ds(base, 64)],
                                 out_hbm.at[pl.ds(dst, 64)], out_sem)

            rank = rank + mski[l]
        n = rank
        return (ptr + n, wb, nfired + n, (n == 16).astype(jnp.int32))

    return lax.while_loop(cond, body, state)


@functools.lru_cache(maxsize=None)
def _make_sc_scan_gather(B: int, D: int, V: int):
    ncols = (V + 127) // 128
    cols_per_w = (ncols + _NW - 1) // _NW
    vtail = (ncols - 1) * 128
    tail_w = V - vtail
    mesh = plsc.VectorSubcoreMesh(core_axis_name="c", subcore_axis_name="s")

    @functools.partial(
        pl.kernel,
        mesh=mesh,
        compiler_params=pltpu.CompilerParams(use_tc_tiling_on_sc=True,
                                             needs_layout_passes=False),
        out_type=jax.ShapeDtypeStruct((B * D,), jnp.float32),
        scratch_types=[
            pltpu.VMEM((_NW + 16,), jnp.int32),
            pltpu.VMEM((_WIN,), jnp.int32),
            pltpu.VMEM((_WIN,), jnp.int32),
            [pltpu.VMEM((D, 128), jnp.float32) for _ in range(_RING)],
            pltpu.VMEM((D, tail_w), jnp.float32),
            pltpu.VMEM((_RSLOTS * 64,), jnp.float32),
            [pltpu.SemaphoreType.DMA for _ in range(_RING)],
            pltpu.SemaphoreType.DMA,
            pltpu.SemaphoreType.DMA,
        ],
    )
    def scan_k(idxs_hbm, ord_hbm, bnds_hbm, tableT_hbm, tail_hbm, out_hbm,
               bnds_v, win_idx_v, win_ord_v, bufs, tailbuf, rbuf,
               col_sems, tail_sem, out_sem):
        wid = lax.axis_index("s") * _NC + lax.axis_index("c")
        col0 = wid * cols_per_w
        pltpu.sync_copy(bnds_hbm, bnds_v)
        bv = bnds_v[pl.ds(wid, 16)]
        lo = bv[0]

        wb0 = pl.multiple_of((lo // 8) * 8, 8)
        pltpu.sync_copy(idxs_hbm.at[pl.ds(wb0, _WIN)], win_idx_v)
        pltpu.sync_copy(ord_hbm.at[pl.ds(wb0, _WIN)], win_ord_v)

        def fire(col, b):
            @pl.when((col < ncols - 1) & (col < col0 + cols_per_w))
            def _():
                src = pl.multiple_of(col * 128, 128)
                pltpu.async_copy(tableT_hbm.at[:, pl.ds(src, 128)], bufs[b],
                                 col_sems[b])

        for b in range(_RING):
            fire(col0 + b, b)

        @pl.when(col0 <= ncols - 1)
        def _():
            @pl.when(ncols - 1 < col0 + cols_per_w)
            def _():
                pltpu.async_copy(tail_hbm, tailbuf, tail_sem)

        def outer(j4, st):
            for b in range(_RING):
                col = col0 + j4 * _RING + b
                live = (col < ncols - 1) & (col < col0 + cols_per_w)

                @pl.when(live)
                def _():
                    pltpu.make_async_copy(
                        tableT_hbm.at[:, pl.ds(0, 128)], bufs[b],
                        col_sems[b]).wait()

                st_new = _walk_run(col, bufs[b], win_idx_v, win_ord_v,
                                   idxs_hbm, ord_hbm, out_hbm, rbuf, out_sem,
                                   (st[0], st[1], st[2],
                                    live.astype(jnp.int32)))
                st = (st_new[0], st_new[1], st_new[2], jnp.int32(1))
                fire(col + _RING, b)
            return st

        n_outer = (cols_per_w + _RING - 1) // _RING
        st = lax.fori_loop(0, n_outer, outer,
                           (lo, wb0, jnp.int32(0), jnp.int32(1)))

        in_range = (col0 <= ncols - 1) & (ncols - 1 < col0 + cols_per_w)

        @pl.when(in_range)
        def _():
            pltpu.make_async_copy(tail_hbm, tailbuf, tail_sem).wait()

        st = lax.cond(
            in_range,
            lambda s: _walk_run(ncols - 1, tailbuf, win_idx_v, win_ord_v,
                                idxs_hbm, ord_hbm, out_hbm, rbuf, out_sem,
                                (s[0], s[1], s[2], jnp.int32(1))),
            lambda s: s, st)

        outstanding = jnp.minimum(st[2], _RSLOTS)

        def drain(i, carry):
            @pl.when(i < outstanding)
            def _():
                pltpu.make_async_copy(out_hbm.at[pl.ds(0, 64)],
                                      rbuf.at[pl.ds(0, 64)], out_sem).wait()
            return carry

        lax.fori_loop(0, _RSLOTS, drain, 0)

    return scan_k


def _tc_body(eT_ref, doseT_ref, w1aT_ref, w1b_ref, b1_ref, w2T_ref, b2_ref,
             w3_ref, b3_ref, out_ref):
    eT = eT_ref[...]
    h = jnp.dot(w1aT_ref[...], eT, preferred_element_type=jnp.float32)
    h = jnp.maximum(h + w1b_ref[...] * doseT_ref[...] + b1_ref[...], 0.0)
    h = jnp.dot(w2T_ref[...], h, preferred_element_type=jnp.float32)
    h = jnp.maximum(h + b2_ref[...], 0.0)
    s = jnp.sum(h * w3_ref[...], axis=0, keepdims=True) + b3_ref[0, 0]
    out_ref[...] = eT * s


@functools.lru_cache(maxsize=None)
def _make_tc_mlp(B: int, D: int, W: int, BLK: int):
    grid = (B // BLK,)
    full = lambda i: (0, 0)
    return pl.pallas_call(
        _tc_body,
        grid=grid,
        in_specs=[
            pl.BlockSpec((D, BLK), lambda i: (0, i)),
            pl.BlockSpec((1, BLK), lambda i: (0, i)),
            pl.BlockSpec((W, D), full),
            pl.BlockSpec((W, 1), full),
            pl.BlockSpec((W, 1), full),
            pl.BlockSpec((W, W), full),
            pl.BlockSpec((W, 1), full),
            pl.BlockSpec((W, 1), full),
            pl.BlockSpec((1, 1), full),
        ],
        out_specs=pl.BlockSpec((D, BLK), lambda i: (0, i)),
        out_shape=jax.ShapeDtypeStruct((D, B), jnp.float32),
    )


def kernel(batch_idx, dose, table, W1, b1, W2, b2, W3, b3):
    B = batch_idx.shape[0]
    V, D = table.shape
    W = W2.shape[0]
    ncols = (V + 127) // 128
    cols_per_w = (ncols + _NW - 1) // _NW
    idx = batch_idx.astype(jnp.int32)

    order = jnp.argsort(idx).astype(jnp.int32)
    idx_sorted = idx[order]
    edges = jnp.arange(_NW + 1, dtype=jnp.int32) * (cols_per_w * 128)
    bnds = jnp.searchsorted(idx_sorted, edges).astype(jnp.int32)
    bnds = jnp.pad(bnds, (0, _NW + 16 - (_NW + 1)))
    sentinel = jnp.full((_WIN,), jnp.int32(2**30))
    idx_pad = jnp.concatenate([idx_sorted, sentinel])
    ord_pad = jnp.concatenate([order, jnp.zeros((_WIN,), jnp.int32)])

    tableT = table.T
    vtail = ((V + 127) // 128 - 1) * 128
    tail = lax.slice(tableT, (0, vtail), (D, V))
    e_flat = _make_sc_scan_gather(B, D, V)(idx_pad, ord_pad, bnds, tableT,
                                           tail)
    eT = e_flat.reshape(B, D).T
    outT = _make_tc_mlp(B, D, W, 2048)(
        eT,
        dose.T,
        W1[:D].T,
        W1[D:D + 1].T,
        b1.reshape(W, 1),
        W2.T,
        b2.reshape(W, 1),
        W3,
        b3.reshape(1, 1),
    )
    return outT.T


# speedup vs baseline: 8.3878x; 1.6524x over previous
import functools

import jax
import jax.numpy as jnp
from jax import lax
from jax.experimental import pallas as pl
from jax.experimental.pallas import tpu as pltpu
from jax.experimental.pallas import tpu_sc as plsc

_NC = 2
_NS = 16
_NW = _NC * _NS
_WIN = 2048
_RING = 3
_COLW = 512
_RSLOTS = 32


def _walk_run(col, buf, win_idx_v, win_ord_v, idxs_hbm, ord_hbm, out_hbm,
              rbuf, out_sem, state):
    def cond(st):
        return st[3] == 1

    def body(st):
        ptr, wb, nfired, _ = st
        need = ptr + 16 > wb + _WIN

        def slide(_):
            wb2 = pl.multiple_of((ptr // 8) * 8, 8)
            pltpu.sync_copy(idxs_hbm.at[pl.ds(wb2, _WIN)], win_idx_v)
            pltpu.sync_copy(ord_hbm.at[pl.ds(wb2, _WIN)], win_ord_v)
            return wb2

        wb = lax.cond(need, slide, lambda _: wb, 0)
        off = ptr - wb
        iv = win_idx_v[pl.ds(off, 16)]
        pv = win_ord_v[pl.ds(off, 16)]
        q = iv >> 9
        mv = iv & 511
        mski = (q == col).astype(jnp.int32)
        rank = jnp.int32(0)
        for l in range(16):
            take = mski[l] == 1
            slot = lax.rem(nfired + rank, jnp.int32(_RSLOTS))

            @pl.when(take)
            def _():
                @pl.when(nfired + rank >= _RSLOTS)
                def _():
                    pltpu.make_async_copy(out_hbm.at[pl.ds(0, 64)],
                                          rbuf.at[pl.ds(0, 64)],
                                          out_sem).wait()
                m16 = jnp.full((16,), mv[l], jnp.int32)
                base = slot * 64
                for k in range(4):
                    fi = lax.iota(jnp.int32, 16) + 16 * k
                    vals = plsc.load_gather(buf, [fi, m16])
                    rbuf[pl.ds(base + 16 * k, 16)] = vals
                dst = pl.multiple_of(pv[l] * 64, 8)
                pltpu.async_copy(rbuf.at[pl.ds(base, 64)],
                                 out_hbm.at[pl.ds(dst, 64)], out_sem)

            rank = rank + mski[l]
        n = rank
        return (ptr + n, wb, nfired + n, (n == 16).astype(jnp.int32))

    return lax.while_loop(cond, body, state)


@functools.lru_cache(maxsize=None)
def _make_sc_scan_gather(B: int, D: int, V: int):
    ncols = (V + _COLW - 1) // _COLW
    cols_per_w = (ncols + _NW - 1) // _NW
    vtail = (ncols - 1) * _COLW
    tail_w = V - vtail
    mesh = plsc.VectorSubcoreMesh(core_axis_name="c", subcore_axis_name="s")

    @functools.partial(
        pl.kernel,
        mesh=mesh,
        compiler_params=pltpu.CompilerParams(use_tc_tiling_on_sc=True,
                                             needs_layout_passes=False),
        out_type=jax.ShapeDtypeStruct((B * D,), jnp.float32),
        scratch_types=[
            pltpu.VMEM((_NW + 16,), jnp.int32),
            pltpu.VMEM((_WIN,), jnp.int32),
            pltpu.VMEM((_WIN,), jnp.int32),
            [pltpu.VMEM((D, _COLW), jnp.float32) for _ in range(_RING)],
            pltpu.VMEM((D, tail_w), jnp.float32),
            pltpu.VMEM((_RSLOTS * 64,), jnp.float32),
            [pltpu.SemaphoreType.DMA for _ in range(_RING)],
            pltpu.SemaphoreType.DMA,
            pltpu.SemaphoreType.DMA,
        ],
    )
    def scan_k(idxs_hbm, ord_hbm, bnds_hbm, tableT_hbm, tail_hbm, out_hbm,
               bnds_v, win_idx_v, win_ord_v, bufs, tailbuf, rbuf,
               col_sems, tail_sem, out_sem):
        wid = lax.axis_index("s") * _NC + lax.axis_index("c")
        col0 = wid * cols_per_w
        pltpu.sync_copy(bnds_hbm, bnds_v)
        bv = bnds_v[pl.ds(wid, 16)]
        lo = bv[0]

        wb0 = pl.multiple_of((lo // 8) * 8, 8)
        pltpu.sync_copy(idxs_hbm.at[pl.ds(wb0, _WIN)], win_idx_v)
        pltpu.sync_copy(ord_hbm.at[pl.ds(wb0, _WIN)], win_ord_v)

        def fire(col, b):
            @pl.when((col < ncols - 1) & (col < col0 + cols_per_w))
            def _():
                src = pl.multiple_of(col * _COLW, 128)
                pltpu.async_copy(tableT_hbm.at[:, pl.ds(src, _COLW)], bufs[b],
                                 col_sems[b])

        for b in range(_RING):
            fire(col0 + b, b)

        @pl.when(col0 <= ncols - 1)
        def _():
            @pl.when(ncols - 1 < col0 + cols_per_w)
            def _():
                pltpu.async_copy(tail_hbm, tailbuf, tail_sem)

        def outer(j4, st):
            for b in range(_RING):
                col = col0 + j4 * _RING + b
                live = (col < ncols - 1) & (col < col0 + cols_per_w)

                @pl.when(live)
                def _():
                    pltpu.make_async_copy(
                        tableT_hbm.at[:, pl.ds(0, _COLW)], bufs[b],
                        col_sems[b]).wait()

                st_new = _walk_run(col, bufs[b], win_idx_v, win_ord_v,
                                   idxs_hbm, ord_hbm, out_hbm, rbuf, out_sem,
                                   (st[0], st[1], st[2],
                                    live.astype(jnp.int32)))
                st = (st_new[0], st_new[1], st_new[2], jnp.int32(1))
                fire(col + _RING, b)
            return st

        n_outer = (cols_per_w + _RING - 1) // _RING
        st = lax.fori_loop(0, n_outer, outer,
                           (lo, wb0, jnp.int32(0), jnp.int32(1)))

        in_range = (col0 <= ncols - 1) & (ncols - 1 < col0 + cols_per_w)

        @pl.when(in_range)
        def _():
            pltpu.make_async_copy(tail_hbm, tailbuf, tail_sem).wait()

        st = lax.cond(
            in_range,
            lambda s: _walk_run(ncols - 1, tailbuf, win_idx_v, win_ord_v,
                                idxs_hbm, ord_hbm, out_hbm, rbuf, out_sem,
                                (s[0], s[1], s[2], jnp.int32(1))),
            lambda s: s, st)

        outstanding = jnp.minimum(st[2], _RSLOTS)

        def drain(i, carry):
            @pl.when(i < outstanding)
            def _():
                pltpu.make_async_copy(out_hbm.at[pl.ds(0, 64)],
                                      rbuf.at[pl.ds(0, 64)], out_sem).wait()
            return carry

        lax.fori_loop(0, _RSLOTS, drain, 0)

    return scan_k


def _tc_body(eT_ref, doseT_ref, w1aT_ref, w1b_ref, b1_ref, w2T_ref, b2_ref,
             w3_ref, b3_ref, out_ref):
    eT = eT_ref[...]
    h = jnp.dot(w1aT_ref[...], eT, preferred_element_type=jnp.float32)
    h = jnp.maximum(h + w1b_ref[...] * doseT_ref[...] + b1_ref[...], 0.0)
    h = jnp.dot(w2T_ref[...], h, preferred_element_type=jnp.float32)
    h = jnp.maximum(h + b2_ref[...], 0.0)
    s = jnp.sum(h * w3_ref[...], axis=0, keepdims=True) + b3_ref[0, 0]
    out_ref[...] = eT * s


@functools.lru_cache(maxsize=None)
def _make_tc_mlp(B: int, D: int, W: int, BLK: int):
    grid = (B // BLK,)
    full = lambda i: (0, 0)
    return pl.pallas_call(
        _tc_body,
        grid=grid,
        in_specs=[
            pl.BlockSpec((D, BLK), lambda i: (0, i)),
            pl.BlockSpec((1, BLK), lambda i: (0, i)),
            pl.BlockSpec((W, D), full),
            pl.BlockSpec((W, 1), full),
            pl.BlockSpec((W, 1), full),
            pl.BlockSpec((W, W), full),
            pl.BlockSpec((W, 1), full),
            pl.BlockSpec((W, 1), full),
            pl.BlockSpec((1, 1), full),
        ],
        out_specs=pl.BlockSpec((D, BLK), lambda i: (0, i)),
        out_shape=jax.ShapeDtypeStruct((D, B), jnp.float32),
    )


def kernel(batch_idx, dose, table, W1, b1, W2, b2, W3, b3):
    B = batch_idx.shape[0]
    V, D = table.shape
    W = W2.shape[0]
    ncols = (V + _COLW - 1) // _COLW
    cols_per_w = (ncols + _NW - 1) // _NW
    idx = batch_idx.astype(jnp.int32)

    order = jnp.argsort(idx).astype(jnp.int32)
    idx_sorted = idx[order]
    edges = jnp.arange(_NW + 1, dtype=jnp.int32) * (cols_per_w * _COLW)
    bnds = jnp.searchsorted(idx_sorted, edges).astype(jnp.int32)
    bnds = jnp.pad(bnds, (0, _NW + 16 - (_NW + 1)))
    sentinel = jnp.full((_WIN,), jnp.int32(2**30))
    idx_pad = jnp.concatenate([idx_sorted, sentinel])
    ord_pad = jnp.concatenate([order, jnp.zeros((_WIN,), jnp.int32)])

    tableT = table.T
    vtail = ((V + _COLW - 1) // _COLW - 1) * _COLW
    tail = lax.slice(tableT, (0, vtail), (D, V))
    e_flat = _make_sc_scan_gather(B, D, V)(idx_pad, ord_pad, bnds, tableT,
                                           tail)
    eT = e_flat.reshape(B, D).T
    outT = _make_tc_mlp(B, D, W, 2048)(
        eT,
        dose.T,
        W1[:D].T,
        W1[D:D + 1].T,
        b1.reshape(W, 1),
        W2.T,
        b2.reshape(W, 1),
        W3,
        b3.reshape(1, 1),
    )
    return outT.T
